# baseline (speedup 1.0000x reference)
import functools

import jax
import jax.numpy as jnp
from jax import lax
from jax.experimental import pallas as pl
from jax.experimental.pallas import tpu as pltpu
import jax.experimental.pallas.tpu_sc as plsc

NC = 2
NS = 16
NW = NC * NS
L = 16
C = 32
DENOM = 16.0


def _rsqrt(v):
    i = plsc.bitcast(v, jnp.int32)
    i = jnp.int32(0x5F3759DF) - lax.shift_right_logical(i, 1)
    y = plsc.bitcast(i, jnp.float32)
    for _ in range(3):
        y = y * (1.5 - 0.5 * v * y * y)
    return y


def _ylm16(rx, ry, rz):
    n2 = rx * rx + ry * ry + rz * rz + 1e-9
    rinv = _rsqrt(n2)
    x = rx * rinv
    y = ry * rinv
    z = rz * rinv
    s3 = 3.0 ** 0.5
    s15 = 15.0 ** 0.5
    s5 = 5.0 ** 0.5
    c3m3 = 0.25 * (70.0 ** 0.5)
    c3m2 = 105.0 ** 0.5
    c3m1 = 0.25 * (42.0 ** 0.5)
    c30 = 0.5 * (7.0 ** 0.5)
    c32 = 0.5 * (105.0 ** 0.5)
    xx = x * x
    yy = y * y
    zz = z * z
    one = x * 0.0 + 1.0
    return [
        one,
        s3 * x, s3 * y, s3 * z,
        s15 * x * y,
        s15 * y * z,
        0.5 * s5 * (3.0 * zz - 1.0),
        s15 * x * z,
        0.5 * s15 * (xx - yy),
        c3m3 * y * (3.0 * xx - yy),
        c3m2 * x * y * z,
        c3m1 * y * (5.0 * zz - 1.0),
        c30 * (5.0 * zz * z - 3.0 * z),
        c3m1 * x * (5.0 * zz - 1.0),
        c32 * z * (xx - yy),
        c3m3 * x * (xx - 3.0 * yy),
    ]


def _edge_prep(x, posx, posy, posz, snd, rcv, n_edges):
    n_nodes = x.shape[0]
    epad = snd.shape[0]
    ew = epad // NW
    nblk = ew // 128
    mesh = plsc.VectorSubcoreMesh(
        core_axis_name="c", subcore_axis_name="s",
        num_cores=NC, num_subcores=NS)

    @functools.partial(
        pl.kernel,
        out_type=(jax.ShapeDtypeStruct((epad, C), jnp.float32),
                  jax.ShapeDtypeStruct((epad * L,), jnp.float32)),
        mesh=mesh,
        compiler_params=pltpu.CompilerParams(
            needs_layout_passes=False, use_tc_tiling_on_sc=False),
        scratch_types=[
            pltpu.VMEM((n_nodes,), jnp.float32),
            pltpu.VMEM((n_nodes,), jnp.float32),
            pltpu.VMEM((n_nodes,), jnp.float32),
            pltpu.VMEM((128,), jnp.int32),
            pltpu.VMEM((128,), jnp.int32),
            pltpu.VMEM((128, C), jnp.float32),
            pltpu.VMEM((128 * L,), jnp.float32),
            pltpu.SemaphoreType.DMA,
        ],
    )
    def prep(x_hbm, px_hbm, py_hbm, pz_hbm, snd_hbm, rcv_hbm,
             fs_hbm, ylm_hbm,
             px_v, py_v, pz_v, snd_v, rcv_v, fsb, ylmb, gsem):
        cid = lax.axis_index("c")
        sid = lax.axis_index("s")
        wid = sid * NC + cid
        base = wid * ew
        pltpu.sync_copy(px_hbm, px_v)
        pltpu.sync_copy(py_hbm, py_v)
        pltpu.sync_copy(pz_hbm, pz_v)
        lanes = lax.iota(jnp.int32, L)

        def body(j, carry):
            off = base + j * 128
            pltpu.sync_copy(snd_hbm.at[pl.ds(off, 128)], snd_v)
            pltpu.sync_copy(rcv_hbm.at[pl.ds(off, 128)], rcv_v)
            desc = pltpu.async_copy(x_hbm.at[snd_v], fsb, gsem)
            for k in range(8):
                sidx = snd_v[pl.ds(k * L, L)]
                ridx = rcv_v[pl.ds(k * L, L)]
                rx = plsc.load_gather(px_v, [ridx]) - plsc.load_gather(px_v, [sidx])
                ry = plsc.load_gather(py_v, [ridx]) - plsc.load_gather(py_v, [sidx])
                rz = plsc.load_gather(pz_v, [ridx]) - plsc.load_gather(pz_v, [sidx])
                comps = _ylm16(rx, ry, rz)
                gidx = off + k * L + lanes
                valid = jnp.where(gidx < n_edges, 1.0, 0.0)
                rowbase = (k * L + lanes) * L
                for ci, comp in enumerate(comps):
                    plsc.store_scatter(ylmb, [rowbase + ci], comp * valid)
            desc.wait()
            pltpu.sync_copy(fsb, fs_hbm.at[pl.ds(off, 128)])
            pltpu.sync_copy(ylmb, ylm_hbm.at[pl.ds(off * L, 128 * L)])
            return carry

        lax.fori_loop(0, nblk, body, 0)

    return prep(x, posx, posy, posz, snd, rcv)


def _accumulate(rcv, fs, ylm, n_pad):
    epad = rcv.shape[0]
    et = epad // NS
    be = 80
    nblk = et // be
    rpt = n_pad // NS
    mesh = plsc.VectorSubcoreMesh(
        core_axis_name="c", subcore_axis_name="s",
        num_cores=NC, num_subcores=NS)

    @functools.partial(
        pl.kernel,
        out_type=jax.ShapeDtypeStruct((4, n_pad, 128), jnp.float32),
        mesh=mesh,
        compiler_params=pltpu.CompilerParams(needs_layout_passes=False),
        scratch_types=[
            pltpu.VMEM((be,), jnp.int32),
            pltpu.VMEM((be,), jnp.int32),
            pltpu.VMEM((be,), jnp.int32),
            pltpu.VMEM((be,), jnp.int32),
            pltpu.VMEM((be, C), jnp.float32),
            pltpu.VMEM((be, C), jnp.float32),
            pltpu.VMEM((be * L,), jnp.float32),
            pltpu.VMEM((be * L,), jnp.float32),
            pltpu.VMEM((be, 128), jnp.float32),
            pltpu.VMEM((be, 128), jnp.float32),
            pltpu.VMEM_SHARED((n_pad, 128), jnp.float32),
            pltpu.SemaphoreType.DMA,
            pltpu.SemaphoreType.DMA,
            pltpu.SemaphoreType.DMA,
            pltpu.SemaphoreType.DMA,
        ],
    )
    def acc(rcv_hbm, fs_hbm, ylm_hbm, out_hbm,
            rcv0, rcv1, rcv2, rcv3, fs_a, fs_b, ylm_a, ylm_b,
            msg_a, msg_b, agg_sh, in_a, in_b, ss_a, ss_b):
        cid = lax.axis_index("c")
        sid = lax.axis_index("s")
        tbase = sid * et
        rows0 = sid * rpt
        lanes = lax.iota(jnp.int32, L)
        zeros = jnp.zeros((L,), jnp.float32)

        def clear_my_rows():
            def zrow(r, carry):
                for cc in range(8):
                    plsc.store_scatter(
                        msg_a, [jnp.full((L,), r, jnp.int32),
                                cc * L + lanes],
                        zeros)
                return carry

            lax.fori_loop(0, be, zrow, 0)
            q, rem = divmod(rpt, be)
            for i in range(q):
                pltpu.sync_copy(msg_a,
                                agg_sh.at[pl.ds(rows0 + i * be, be)])
            if rem:
                pltpu.sync_copy(msg_a.at[pl.ds(0, rem)],
                                agg_sh.at[pl.ds(rows0 + q * be, rem)])

        clear_my_rows()
        plsc.subcore_barrier()

        for p in range(2):
            g = cid * 2 + p

            def start_in(j, rcv_v, fs_v, ylm_v, sem):
                off = tbase + j * be
                return (
                    pltpu.async_copy(rcv_hbm.at[pl.ds(off, be)],
                                     rcv_v, sem),
                    pltpu.async_copy(fs_hbm.at[pl.ds(off, be)],
                                     fs_v, sem),
                    pltpu.async_copy(ylm_hbm.at[pl.ds(off * L, be * L)],
                                     ylm_v, sem),
                )

            def wait_in(descs):
                for d in descs:
                    d.wait()

            def compute(fs_v, ylm_v, msg_v):
                @plsc.parallel_loop(0, be, unroll=4)
                def edge(e):
                    yrow = ylm_v[pl.ds(e * L, L)]
                    fsrow = fs_v[e, pl.ds(cid * L, L)]
                    for c8 in range(4):
                        s = fsrow[p * 8 + c8]
                        msg_v[e, pl.ds(c8 * L, L)] = s * yrow

            def scat(msg_v, rcv_v, sem):
                return pltpu.async_copy(msg_v,
                                        agg_sh.at[pl.ds(rows0, be)],
                                        sem, add=False)

            def blk(jj, carry):
                j0 = jj * 4
                d0 = start_in(j0, rcv0, fs_a, ylm_a, in_a)
                d1 = start_in(j0 + 1, rcv1, fs_b, ylm_b, in_b)
                wait_in(d0)
                compute(fs_a, ylm_a, msg_a)
                s0 = scat(msg_a, rcv0, ss_a)
                d2 = start_in(j0 + 2, rcv2, fs_a, ylm_a, in_a)
                wait_in(d1)
                compute(fs_b, ylm_b, msg_b)
                s1 = scat(msg_b, rcv1, ss_b)
                d3 = start_in(j0 + 3, rcv3, fs_b, ylm_b, in_b)
                s0.wait()
                wait_in(d2)
                compute(fs_a, ylm_a, msg_a)
                s2 = scat(msg_a, rcv2, ss_a)
                s1.wait()
                wait_in(d3)
                compute(fs_b, ylm_b, msg_b)
                s3 = scat(msg_b, rcv3, ss_b)
                s2.wait()
                s3.wait()
                return carry

            lax.fori_loop(0, nblk // 4, blk, 0)
            plsc.subcore_barrier()
            q, rem = divmod(rpt, 128)
            for i in range(q):
                pltpu.sync_copy(agg_sh.at[pl.ds(rows0 + i * 128, 128)],
                                out_hbm.at[g, pl.ds(rows0 + i * 128, 128)])
            if rem:
                pltpu.sync_copy(agg_sh.at[pl.ds(rows0 + q * 128, rem)],
                                out_hbm.at[g, pl.ds(rows0 + q * 128, rem)])
            if p == 0:
                clear_my_rows()
            plsc.subcore_barrier()

    return acc(rcv, fs, ylm)


def _build_b(w0, w1, w2, w3):
    s = (1.0 / DENOM) * (1.0 / (C ** 0.5))
    b = jnp.zeros((4, 128, 512), jnp.float32)
    for moff, d, off, w in ((0, 1, 0, w0), (1, 3, 32, w1),
                            (4, 5, 128, w2), (9, 7, 288, w3)):
        wg = (s * w).reshape(4, 8, C)
        for i in range(d):
            rows = jnp.arange(8) * L + moff + i
            cols = off + jnp.arange(C) * d + i
            b = b.at[:, rows[:, None], cols[None, :]].set(wg)
    return b


def _finish(agg4, x, bmat, wsc):
    n_nodes = x.shape[0]
    blk = 400
    grid = (n_nodes // blk,)
    inv = 1.0 / (C ** 0.5)

    def body(agg_ref, b_ref, x_ref, wsc_ref, out_ref):
        acc = jnp.zeros((blk, 512), jnp.float32)
        for p in range(4):
            acc = acc + jnp.dot(agg_ref[p], b_ref[p],
                                preferred_element_type=jnp.float32)
        sc0 = jnp.dot(x_ref[...], wsc_ref[...],
                      preferred_element_type=jnp.float32) * inv
        out_ref[...] = acc
        out_ref[:, 0:C] = acc[:, 0:C] + sc0

    return pl.pallas_call(
        body,
        grid=grid,
        in_specs=[
            pl.BlockSpec((4, blk, 128), lambda i: (0, i, 0)),
            pl.BlockSpec((4, 128, 512), lambda i: (0, 0, 0)),
            pl.BlockSpec((blk, C), lambda i: (i, 0)),
            pl.BlockSpec((C, C), lambda i: (0, 0)),
        ],
        out_specs=pl.BlockSpec((blk, 512), lambda i: (i, 0)),
        out_shape=jax.ShapeDtypeStruct((n_nodes, 512), jnp.float32),
    )(agg4, bmat, x, wsc)


def kernel(x, positions, senders, receivers, W0, W1, W2, W3, Wsc):
    n_nodes = x.shape[0]
    n_edges = senders.shape[0]
    quantum = NW * 128
    epad = ((n_edges + quantum - 1) // quantum) * quantum
    snd = jnp.pad(senders.astype(jnp.int32), (0, epad - n_edges))
    rcv = jnp.pad(receivers.astype(jnp.int32), (0, epad - n_edges))
    posx = positions[:, 0]
    posy = positions[:, 1]
    posz = positions[:, 2]
    fs, ylm = _edge_prep(x, posx, posy, posz, snd, rcv, n_edges)
    n_pad = ((n_nodes + 8 * NS - 1) // (8 * NS)) * (8 * NS)
    agg4 = _accumulate(rcv, fs, ylm, n_pad)
    bmat = _build_b(W0, W1, W2, W3)
    return _finish(agg4, x, bmat, Wsc)


# speedup vs baseline: 1.4359x; 1.4359x over previous
import functools

import jax
import jax.numpy as jnp
from jax import lax
from jax.experimental import pallas as pl
from jax.experimental.pallas import tpu as pltpu
import jax.experimental.pallas.tpu_sc as plsc

NC = 2
NS = 16
NW = NC * NS
L = 16
C = 32
DENOM = 16.0


def _rsqrt(v):
    i = plsc.bitcast(v, jnp.int32)
    i = jnp.int32(0x5F3759DF) - lax.shift_right_logical(i, 1)
    y = plsc.bitcast(i, jnp.float32)
    for _ in range(3):
        y = y * (1.5 - 0.5 * v * y * y)
    return y


def _ylm16(rx, ry, rz):
    n2 = rx * rx + ry * ry + rz * rz + 1e-9
    rinv = _rsqrt(n2)
    x = rx * rinv
    y = ry * rinv
    z = rz * rinv
    s3 = 3.0 ** 0.5
    s15 = 15.0 ** 0.5
    s5 = 5.0 ** 0.5
    c3m3 = 0.25 * (70.0 ** 0.5)
    c3m2 = 105.0 ** 0.5
    c3m1 = 0.25 * (42.0 ** 0.5)
    c30 = 0.5 * (7.0 ** 0.5)
    c32 = 0.5 * (105.0 ** 0.5)
    xx = x * x
    yy = y * y
    zz = z * z
    one = x * 0.0 + 1.0
    return [
        one,
        s3 * x, s3 * y, s3 * z,
        s15 * x * y,
        s15 * y * z,
        0.5 * s5 * (3.0 * zz - 1.0),
        s15 * x * z,
        0.5 * s15 * (xx - yy),
        c3m3 * y * (3.0 * xx - yy),
        c3m2 * x * y * z,
        c3m1 * y * (5.0 * zz - 1.0),
        c30 * (5.0 * zz * z - 3.0 * z),
        c3m1 * x * (5.0 * zz - 1.0),
        c32 * z * (xx - yy),
        c3m3 * x * (xx - 3.0 * yy),
    ]


def _edge_prep(x, posx, posy, posz, snd, rcv, n_edges):
    n_nodes = x.shape[0]
    epad = snd.shape[0]
    ew = epad // NW
    nblk = ew // 128
    mesh = plsc.VectorSubcoreMesh(
        core_axis_name="c", subcore_axis_name="s",
        num_cores=NC, num_subcores=NS)

    @functools.partial(
        pl.kernel,
        out_type=(jax.ShapeDtypeStruct((epad, C), jnp.float32),
                  jax.ShapeDtypeStruct((epad * L,), jnp.float32)),
        mesh=mesh,
        compiler_params=pltpu.CompilerParams(
            needs_layout_passes=False, use_tc_tiling_on_sc=False),
        scratch_types=[
            pltpu.VMEM((n_nodes,), jnp.float32),
            pltpu.VMEM((n_nodes,), jnp.float32),
            pltpu.VMEM((n_nodes,), jnp.float32),
            pltpu.VMEM((128,), jnp.int32),
            pltpu.VMEM((128,), jnp.int32),
            pltpu.VMEM((128,), jnp.int32),
            pltpu.VMEM((128,), jnp.int32),
            pltpu.VMEM((128, C), jnp.float32),
            pltpu.VMEM((128, C), jnp.float32),
            pltpu.VMEM((128 * L,), jnp.float32),
            pltpu.VMEM((128 * L,), jnp.float32),
            pltpu.SemaphoreType.DMA,
            pltpu.SemaphoreType.DMA,
            pltpu.SemaphoreType.DMA,
            pltpu.SemaphoreType.DMA,
            pltpu.SemaphoreType.DMA,
            pltpu.SemaphoreType.DMA,
        ],
    )
    def prep(x_hbm, px_hbm, py_hbm, pz_hbm, snd_hbm, rcv_hbm,
             fs_hbm, ylm_hbm,
             px_v, py_v, pz_v, snd_a, rcv_a, snd_b, rcv_b,
             fs_a, fs_b, ylm_a, ylm_b,
             in_a, in_b, g_a, g_b, o_a, o_b):
        cid = lax.axis_index("c")
        sid = lax.axis_index("s")
        wid = sid * NC + cid
        base = wid * ew
        pltpu.sync_copy(px_hbm, px_v)
        pltpu.sync_copy(py_hbm, py_v)
        pltpu.sync_copy(pz_hbm, pz_v)
        lanes = lax.iota(jnp.int32, L)

        def start_in(j, snd_v, rcv_v, sem):
            off = base + j * 128
            return (
                pltpu.async_copy(snd_hbm.at[pl.ds(off, 128)],
                                 snd_v, sem),
                pltpu.async_copy(rcv_hbm.at[pl.ds(off, 128)],
                                 rcv_v, sem),
            )

        def compute(off, snd_v, rcv_v, ylmb):
            for k in range(8):
                sidx = snd_v[pl.ds(k * L, L)]
                ridx = rcv_v[pl.ds(k * L, L)]
                rx = plsc.load_gather(px_v, [ridx]) - plsc.load_gather(px_v, [sidx])
                ry = plsc.load_gather(py_v, [ridx]) - plsc.load_gather(py_v, [sidx])
                rz = plsc.load_gather(pz_v, [ridx]) - plsc.load_gather(pz_v, [sidx])
                comps = _ylm16(rx, ry, rz)
                gidx = off + k * L + lanes
                valid = jnp.where(gidx < n_edges, 1.0, 0.0)
                rowbase = (k * L + lanes) * L
                for ci, comp in enumerate(comps):
                    plsc.store_scatter(ylmb, [rowbase + ci], comp * valid)

        def start_out(j, fsb, ylmb, sem):
            off = base + j * 128
            return (
                pltpu.async_copy(fsb, fs_hbm.at[pl.ds(off, 128)], sem),
                pltpu.async_copy(ylmb,
                                 ylm_hbm.at[pl.ds(off * L, 128 * L)],
                                 sem),
            )

        def body(jj, carry):
            j0 = jj * 2
            d0 = start_in(j0, snd_a, rcv_a, in_a)
            d1 = start_in(j0 + 1, snd_b, rcv_b, in_b)
            for d in d0:
                d.wait()
            ga = pltpu.async_copy(x_hbm.at[snd_a], fs_a, g_a)
            compute(base + j0 * 128, snd_a, rcv_a, ylm_a)
            ga.wait()
            oa = start_out(j0, fs_a, ylm_a, o_a)
            for d in d1:
                d.wait()
            gb = pltpu.async_copy(x_hbm.at[snd_b], fs_b, g_b)
            compute(base + (j0 + 1) * 128, snd_b, rcv_b, ylm_b)
            gb.wait()
            ob = start_out(j0 + 1, fs_b, ylm_b, o_b)
            for d in oa:
                d.wait()
            for d in ob:
                d.wait()
            return carry

        lax.fori_loop(0, nblk // 2, body, 0)

    return prep(x, posx, posy, posz, snd, rcv)


def _accumulate(rcv, fs, ylm, n_pad):
    epad = rcv.shape[0]
    et = epad // NS
    be = 80
    nblk = et // be
    rpt = n_pad // NS
    mesh = plsc.VectorSubcoreMesh(
        core_axis_name="c", subcore_axis_name="s",
        num_cores=NC, num_subcores=NS)

    @functools.partial(
        pl.kernel,
        out_type=jax.ShapeDtypeStruct((4, n_pad, 128), jnp.float32),
        mesh=mesh,
        compiler_params=pltpu.CompilerParams(needs_layout_passes=False),
        scratch_types=[
            pltpu.VMEM((be,), jnp.int32),
            pltpu.VMEM((be,), jnp.int32),
            pltpu.VMEM((be,), jnp.int32),
            pltpu.VMEM((be,), jnp.int32),
            pltpu.VMEM((be, C), jnp.float32),
            pltpu.VMEM((be, C), jnp.float32),
            pltpu.VMEM((be * L,), jnp.float32),
            pltpu.VMEM((be * L,), jnp.float32),
            pltpu.VMEM((be, 128), jnp.float32),
            pltpu.VMEM((be, 128), jnp.float32),
            pltpu.VMEM_SHARED((n_pad, 128), jnp.float32),
            pltpu.SemaphoreType.DMA,
            pltpu.SemaphoreType.DMA,
            pltpu.SemaphoreType.DMA,
            pltpu.SemaphoreType.DMA,
        ],
    )
    def acc(rcv_hbm, fs_hbm, ylm_hbm, out_hbm,
            rcv0, rcv1, rcv2, rcv3, fs_a, fs_b, ylm_a, ylm_b,
            msg_a, msg_b, agg_sh, in_a, in_b, ss_a, ss_b):
        cid = lax.axis_index("c")
        sid = lax.axis_index("s")
        tbase = sid * et
        rows0 = sid * rpt
        lanes = lax.iota(jnp.int32, L)
        zeros = jnp.zeros((L,), jnp.float32)

        def clear_my_rows():
            def zrow(r, carry):
                for cc in range(8):
                    plsc.store_scatter(
                        msg_a, [jnp.full((L,), r, jnp.int32),
                                cc * L + lanes],
                        zeros)
                return carry

            lax.fori_loop(0, be, zrow, 0)
            q, rem = divmod(rpt, be)
            for i in range(q):
                pltpu.sync_copy(msg_a,
                                agg_sh.at[pl.ds(rows0 + i * be, be)])
            if rem:
                pltpu.sync_copy(msg_a.at[pl.ds(0, rem)],
                                agg_sh.at[pl.ds(rows0 + q * be, rem)])

        clear_my_rows()
        plsc.subcore_barrier()

        for p in range(2):
            g = cid * 2 + p

            def start_in(j, rcv_v, fs_v, ylm_v, sem):
                off = tbase + j * be
                return (
                    pltpu.async_copy(rcv_hbm.at[pl.ds(off, be)],
                                     rcv_v, sem),
                    pltpu.async_copy(fs_hbm.at[pl.ds(off, be)],
                                     fs_v, sem),
                    pltpu.async_copy(ylm_hbm.at[pl.ds(off * L, be * L)],
                                     ylm_v, sem),
                )

            def wait_in(descs):
                for d in descs:
                    d.wait()

            def compute(fs_v, ylm_v, msg_v):
                @plsc.parallel_loop(0, be, unroll=4)
                def edge(e):
                    yrow = ylm_v[pl.ds(e * L, L)]
                    fsrow = fs_v[e, pl.ds(cid * L, L)]
                    for c8 in range(8):
                        s = fsrow[p * 8 + c8]
                        msg_v[e, pl.ds(c8 * L, L)] = s * yrow

            def scat(msg_v, rcv_v, sem):
                return pltpu.async_copy(msg_v, agg_sh.at[rcv_v],
                                        sem, add=True)

            def blk(jj, carry):
                j0 = jj * 4
                d0 = start_in(j0, rcv0, fs_a, ylm_a, in_a)
                d1 = start_in(j0 + 1, rcv1, fs_b, ylm_b, in_b)
                wait_in(d0)
                compute(fs_a, ylm_a, msg_a)
                s0 = scat(msg_a, rcv0, ss_a)
                d2 = start_in(j0 + 2, rcv2, fs_a, ylm_a, in_a)
                wait_in(d1)
                compute(fs_b, ylm_b, msg_b)
                s1 = scat(msg_b, rcv1, ss_b)
                d3 = start_in(j0 + 3, rcv3, fs_b, ylm_b, in_b)
                s0.wait()
                wait_in(d2)
                compute(fs_a, ylm_a, msg_a)
                s2 = scat(msg_a, rcv2, ss_a)
                s1.wait()
                wait_in(d3)
                compute(fs_b, ylm_b, msg_b)
                s3 = scat(msg_b, rcv3, ss_b)
                s2.wait()
                s3.wait()
                return carry

            lax.fori_loop(0, nblk // 4, blk, 0)
            plsc.subcore_barrier()
            q, rem = divmod(rpt, 128)
            for i in range(q):
                pltpu.sync_copy(agg_sh.at[pl.ds(rows0 + i * 128, 128)],
                                out_hbm.at[g, pl.ds(rows0 + i * 128, 128)])
            if rem:
                pltpu.sync_copy(agg_sh.at[pl.ds(rows0 + q * 128, rem)],
                                out_hbm.at[g, pl.ds(rows0 + q * 128, rem)])
            if p == 0:
                clear_my_rows()
            plsc.subcore_barrier()

    return acc(rcv, fs, ylm)


def _build_b(w0, w1, w2, w3):
    import numpy as np
    s = (1.0 / DENOM) * (1.0 / (C ** 0.5))
    parts = []
    for moff, d, w in ((0, 1, w0), (1, 3, w1), (4, 5, w2), (9, 7, w3)):
        e = np.zeros((L, d), np.float32)
        for i in range(d):
            e[moff + i, i] = 1.0
        wg = (s * w).reshape(4, 8, C)
        parts.append(
            jnp.einsum('gco,ri->gcroi', wg, jnp.asarray(e))
            .reshape(4, 128, C * d))
    return jnp.concatenate(parts, axis=-1)


def _finish(agg4, x, bmat, wsc):
    n_nodes = x.shape[0]
    blk = 400
    grid = (n_nodes // blk,)
    inv = 1.0 / (C ** 0.5)

    def body(agg_ref, b_ref, x_ref, wsc_ref, out_ref):
        acc = jnp.zeros((blk, 512), jnp.float32)
        for p in range(4):
            acc = acc + jnp.dot(agg_ref[p], b_ref[p],
                                preferred_element_type=jnp.float32)
        sc0 = jnp.dot(x_ref[...], wsc_ref[...],
                      preferred_element_type=jnp.float32) * inv
        out_ref[...] = acc
        out_ref[:, 0:C] = acc[:, 0:C] + sc0

    return pl.pallas_call(
        body,
        grid=grid,
        in_specs=[
            pl.BlockSpec((4, blk, 128), lambda i: (0, i, 0)),
            pl.BlockSpec((4, 128, 512), lambda i: (0, 0, 0)),
            pl.BlockSpec((blk, C), lambda i: (i, 0)),
            pl.BlockSpec((C, C), lambda i: (0, 0)),
        ],
        out_specs=pl.BlockSpec((blk, 512), lambda i: (i, 0)),
        out_shape=jax.ShapeDtypeStruct((n_nodes, 512), jnp.float32),
    )(agg4, bmat, x, wsc)


def kernel(x, positions, senders, receivers, W0, W1, W2, W3, Wsc):
    n_nodes = x.shape[0]
    n_edges = senders.shape[0]
    quantum = NW * 128
    epad = ((n_edges + quantum - 1) // quantum) * quantum
    snd = jnp.pad(senders.astype(jnp.int32), (0, epad - n_edges))
    rcv = jnp.pad(receivers.astype(jnp.int32), (0, epad - n_edges))
    posx = positions[:, 0]
    posy = positions[:, 1]
    posz = positions[:, 2]
    fs, ylm = _edge_prep(x, posx, posy, posz, snd, rcv, n_edges)
    n_pad = ((n_nodes + 8 * NS - 1) // (8 * NS)) * (8 * NS)
    agg4 = _accumulate(rcv, fs, ylm, n_pad)
    bmat = _build_b(W0, W1, W2, W3)
    return _finish(agg4, x, bmat, Wsc)


# speedup vs baseline: 1.8198x; 1.2673x over previous
import functools

import jax
import jax.numpy as jnp
from jax import lax
from jax.experimental import pallas as pl
from jax.experimental.pallas import tpu as pltpu
import jax.experimental.pallas.tpu_sc as plsc

NC = 2
NS = 16
NW = NC * NS
L = 16
C = 32
DENOM = 16.0


def _rsqrt(v):
    i = plsc.bitcast(v, jnp.int32)
    i = jnp.int32(0x5F3759DF) - lax.shift_right_logical(i, 1)
    y = plsc.bitcast(i, jnp.float32)
    for _ in range(3):
        y = y * (1.5 - 0.5 * v * y * y)
    return y


def _ylm16(rx, ry, rz):
    n2 = rx * rx + ry * ry + rz * rz + 1e-9
    rinv = _rsqrt(n2)
    x = rx * rinv
    y = ry * rinv
    z = rz * rinv
    s3 = 3.0 ** 0.5
    s15 = 15.0 ** 0.5
    s5 = 5.0 ** 0.5
    c3m3 = 0.25 * (70.0 ** 0.5)
    c3m2 = 105.0 ** 0.5
    c3m1 = 0.25 * (42.0 ** 0.5)
    c30 = 0.5 * (7.0 ** 0.5)
    c32 = 0.5 * (105.0 ** 0.5)
    xx = x * x
    yy = y * y
    zz = z * z
    one = x * 0.0 + 1.0
    return [
        one,
        s3 * x, s3 * y, s3 * z,
        s15 * x * y,
        s15 * y * z,
        0.5 * s5 * (3.0 * zz - 1.0),
        s15 * x * z,
        0.5 * s15 * (xx - yy),
        c3m3 * y * (3.0 * xx - yy),
        c3m2 * x * y * z,
        c3m1 * y * (5.0 * zz - 1.0),
        c30 * (5.0 * zz * z - 3.0 * z),
        c3m1 * x * (5.0 * zz - 1.0),
        c32 * z * (xx - yy),
        c3m3 * x * (xx - 3.0 * yy),
    ]


def _edge_prep(x, posx, posy, posz, snd, rcv, n_edges):
    n_nodes = x.shape[0]
    epad = snd.shape[0]
    ew = epad // NW
    nblk = ew // 128
    mesh = plsc.VectorSubcoreMesh(
        core_axis_name="c", subcore_axis_name="s",
        num_cores=NC, num_subcores=NS)

    @functools.partial(
        pl.kernel,
        out_type=(jax.ShapeDtypeStruct((epad * C,), jnp.float32),
                  jax.ShapeDtypeStruct((epad * L,), jnp.float32)),
        mesh=mesh,
        compiler_params=pltpu.CompilerParams(
            needs_layout_passes=False, use_tc_tiling_on_sc=False),
        scratch_types=[
            pltpu.VMEM((n_nodes,), jnp.float32),
            pltpu.VMEM((n_nodes,), jnp.float32),
            pltpu.VMEM((n_nodes,), jnp.float32),
            pltpu.VMEM((128,), jnp.int32),
            pltpu.VMEM((128,), jnp.int32),
            pltpu.VMEM((128,), jnp.int32),
            pltpu.VMEM((128,), jnp.int32),
            pltpu.VMEM((128, C), jnp.float32),
            pltpu.VMEM((128, C), jnp.float32),
            pltpu.VMEM((128 * C,), jnp.float32),
            pltpu.VMEM((128 * C,), jnp.float32),
            pltpu.VMEM((128 * L,), jnp.float32),
            pltpu.VMEM((128 * L,), jnp.float32),
            pltpu.SemaphoreType.DMA,
            pltpu.SemaphoreType.DMA,
            pltpu.SemaphoreType.DMA,
            pltpu.SemaphoreType.DMA,
            pltpu.SemaphoreType.DMA,
            pltpu.SemaphoreType.DMA,
        ],
    )
    def prep(x_hbm, px_hbm, py_hbm, pz_hbm, snd_hbm, rcv_hbm,
             fs_hbm, ylm_hbm,
             px_v, py_v, pz_v, snd_a, rcv_a, snd_b, rcv_b,
             fs_a, fs_b, fsf_a, fsf_b, ylm_a, ylm_b,
             in_a, in_b, g_a, g_b, o_a, o_b):
        cid = lax.axis_index("c")
        sid = lax.axis_index("s")
        wid = sid * NC + cid
        base = wid * ew
        pltpu.sync_copy(px_hbm, px_v)
        pltpu.sync_copy(py_hbm, py_v)
        pltpu.sync_copy(pz_hbm, pz_v)
        lanes = lax.iota(jnp.int32, L)

        def start_in(j, snd_v, rcv_v, sem):
            off = base + j * 128
            return (
                pltpu.async_copy(snd_hbm.at[pl.ds(off, 128)],
                                 snd_v, sem),
                pltpu.async_copy(rcv_hbm.at[pl.ds(off, 128)],
                                 rcv_v, sem),
            )

        def compute(off, snd_v, rcv_v, ylmb):
            for k in range(8):
                sidx = snd_v[pl.ds(k * L, L)]
                ridx = rcv_v[pl.ds(k * L, L)]
                rx = plsc.load_gather(px_v, [ridx]) - plsc.load_gather(px_v, [sidx])
                ry = plsc.load_gather(py_v, [ridx]) - plsc.load_gather(py_v, [sidx])
                rz = plsc.load_gather(pz_v, [ridx]) - plsc.load_gather(pz_v, [sidx])
                comps = _ylm16(rx, ry, rz)
                gidx = off + k * L + lanes
                valid = jnp.where(gidx < n_edges, 1.0, 0.0)
                rowbase = (k * L + lanes) * L
                for ci, comp in enumerate(comps):
                    plsc.store_scatter(ylmb, [rowbase + ci], comp * valid)

        def repack(fsb, fsf):
            def row(e, carry):
                fsf[pl.ds(e * C, L)] = fsb[e, pl.ds(0, L)]
                fsf[pl.ds(e * C + L, L)] = fsb[e, pl.ds(L, L)]
                return carry

            lax.fori_loop(0, 128, row, 0)

        def start_out(j, fsf, ylmb, sem):
            off = base + j * 128
            return (
                pltpu.async_copy(fsf, fs_hbm.at[pl.ds(off * C, 128 * C)],
                                 sem),
                pltpu.async_copy(ylmb,
                                 ylm_hbm.at[pl.ds(off * L, 128 * L)],
                                 sem),
            )

        def body(jj, carry):
            j0 = jj * 2
            d0 = start_in(j0, snd_a, rcv_a, in_a)
            d1 = start_in(j0 + 1, snd_b, rcv_b, in_b)
            for d in d0:
                d.wait()
            ga = pltpu.async_copy(x_hbm.at[snd_a], fs_a, g_a)
            compute(base + j0 * 128, snd_a, rcv_a, ylm_a)
            ga.wait()
            repack(fs_a, fsf_a)
            oa = start_out(j0, fsf_a, ylm_a, o_a)
            for d in d1:
                d.wait()
            gb = pltpu.async_copy(x_hbm.at[snd_b], fs_b, g_b)
            compute(base + (j0 + 1) * 128, snd_b, rcv_b, ylm_b)
            gb.wait()
            repack(fs_b, fsf_b)
            ob = start_out(j0 + 1, fsf_b, ylm_b, o_b)
            for d in oa:
                d.wait()
            for d in ob:
                d.wait()
            return carry

        lax.fori_loop(0, nblk // 2, body, 0)

    return prep(x, posx, posy, posz, snd, rcv)


def _accumulate(rcv, fs, ylm, n_pad):
    epad = rcv.shape[0]
    et = epad // NS
    be = 80
    nblk = et // be
    rpt = n_pad // NS
    mesh = plsc.VectorSubcoreMesh(
        core_axis_name="c", subcore_axis_name="s",
        num_cores=NC, num_subcores=NS)

    @functools.partial(
        pl.kernel,
        out_type=jax.ShapeDtypeStruct((4, n_pad, 128), jnp.float32),
        mesh=mesh,
        compiler_params=pltpu.CompilerParams(needs_layout_passes=False),
        scratch_types=[
            pltpu.VMEM((be,), jnp.int32),
            pltpu.VMEM((be,), jnp.int32),
            pltpu.VMEM((be,), jnp.int32),
            pltpu.VMEM((be,), jnp.int32),
            pltpu.VMEM((be * C,), jnp.float32),
            pltpu.VMEM((be * C,), jnp.float32),
            pltpu.VMEM((be * L,), jnp.float32),
            pltpu.VMEM((be * L,), jnp.float32),
            pltpu.VMEM((be, 128), jnp.float32),
            pltpu.VMEM((be, 128), jnp.float32),
            pltpu.VMEM_SHARED((n_pad, 128), jnp.float32),
            pltpu.SemaphoreType.DMA,
            pltpu.SemaphoreType.DMA,
            pltpu.SemaphoreType.DMA,
            pltpu.SemaphoreType.DMA,
        ],
    )
    def acc(rcv_hbm, fs_hbm, ylm_hbm, out_hbm,
            rcv0, rcv1, rcv2, rcv3, fs_a, fs_b, ylm_a, ylm_b,
            msg_a, msg_b, agg_sh, in_a, in_b, ss_a, ss_b):
        cid = lax.axis_index("c")
        sid = lax.axis_index("s")
        tbase = sid * et
        rows0 = sid * rpt
        lanes = lax.iota(jnp.int32, L)
        zeros = jnp.zeros((L,), jnp.float32)

        def clear_my_rows():
            def zrow(r, carry):
                for cc in range(8):
                    plsc.store_scatter(
                        msg_a, [jnp.full((L,), r, jnp.int32),
                                cc * L + lanes],
                        zeros)
                return carry

            lax.fori_loop(0, be, zrow, 0)
            q, rem = divmod(rpt, be)
            for i in range(q):
                pltpu.sync_copy(msg_a,
                                agg_sh.at[pl.ds(rows0 + i * be, be)])
            if rem:
                pltpu.sync_copy(msg_a.at[pl.ds(0, rem)],
                                agg_sh.at[pl.ds(rows0 + q * be, rem)])

        clear_my_rows()
        plsc.subcore_barrier()

        for p in range(2):
            g = cid * 2 + p

            def start_in(j, rcv_v, fs_v, ylm_v, sem):
                off = tbase + j * be
                return (
                    pltpu.async_copy(rcv_hbm.at[pl.ds(off, be)],
                                     rcv_v, sem),
                    pltpu.async_copy(fs_hbm.at[pl.ds(off * C, be * C)],
                                     fs_v, sem),
                    pltpu.async_copy(ylm_hbm.at[pl.ds(off * L, be * L)],
                                     ylm_v, sem),
                )

            def wait_in(descs):
                for d in descs:
                    d.wait()

            def compute(fs_v, ylm_v, msg_v):
                @plsc.parallel_loop(0, be, unroll=4)
                def edge(e):
                    yrow = ylm_v[pl.ds(e * L, L)]
                    fsrow = fs_v[pl.ds(e * C + cid * L, L)]
                    for c8 in range(8):
                        s = fsrow[p * 8 + c8]
                        msg_v[e, pl.ds(c8 * L, L)] = s * yrow

            def scat(msg_v, rcv_v, sem):
                return pltpu.async_copy(msg_v, agg_sh.at[rcv_v],
                                        sem, add=True)

            def blk(jj, carry):
                j0 = jj * 4
                d0 = start_in(j0, rcv0, fs_a, ylm_a, in_a)
                d1 = start_in(j0 + 1, rcv1, fs_b, ylm_b, in_b)
                wait_in(d0)
                compute(fs_a, ylm_a, msg_a)
                s0 = scat(msg_a, rcv0, ss_a)
                d2 = start_in(j0 + 2, rcv2, fs_a, ylm_a, in_a)
                wait_in(d1)
                compute(fs_b, ylm_b, msg_b)
                s1 = scat(msg_b, rcv1, ss_b)
                d3 = start_in(j0 + 3, rcv3, fs_b, ylm_b, in_b)
                s0.wait()
                wait_in(d2)
                compute(fs_a, ylm_a, msg_a)
                s2 = scat(msg_a, rcv2, ss_a)
                s1.wait()
                wait_in(d3)
                compute(fs_b, ylm_b, msg_b)
                s3 = scat(msg_b, rcv3, ss_b)
                s2.wait()
                s3.wait()
                return carry

            lax.fori_loop(0, nblk // 4, blk, 0)
            plsc.subcore_barrier()
            q, rem = divmod(rpt, 128)
            for i in range(q):
                pltpu.sync_copy(agg_sh.at[pl.ds(rows0 + i * 128, 128)],
                                out_hbm.at[g, pl.ds(rows0 + i * 128, 128)])
            if rem:
                pltpu.sync_copy(agg_sh.at[pl.ds(rows0 + q * 128, rem)],
                                out_hbm.at[g, pl.ds(rows0 + q * 128, rem)])
            if p == 0:
                clear_my_rows()
            plsc.subcore_barrier()

    return acc(rcv, fs, ylm)


def _build_b(w0, w1, w2, w3):
    import numpy as np
    s = (1.0 / DENOM) * (1.0 / (C ** 0.5))
    parts = []
    for moff, d, w in ((0, 1, w0), (1, 3, w1), (4, 5, w2), (9, 7, w3)):
        e = np.zeros((L, d), np.float32)
        for i in range(d):
            e[moff + i, i] = 1.0
        wg = (s * w).reshape(4, 8, C)
        parts.append(
            jnp.einsum('gco,ri->gcroi', wg, jnp.asarray(e))
            .reshape(4, 128, C * d))
    return jnp.concatenate(parts, axis=-1)


def _finish(agg4, x, bmat, wsc):
    n_nodes = x.shape[0]
    blk = 400
    grid = (n_nodes // blk,)
    inv = 1.0 / (C ** 0.5)

    def body(agg_ref, b_ref, x_ref, wsc_ref, out_ref):
        acc = jnp.zeros((blk, 512), jnp.float32)
        for p in range(4):
            acc = acc + jnp.dot(agg_ref[p], b_ref[p],
                                preferred_element_type=jnp.float32)
        sc0 = jnp.dot(x_ref[...], wsc_ref[...],
                      preferred_element_type=jnp.float32) * inv
        out_ref[...] = acc
        out_ref[:, 0:C] = acc[:, 0:C] + sc0

    return pl.pallas_call(
        body,
        grid=grid,
        in_specs=[
            pl.BlockSpec((4, blk, 128), lambda i: (0, i, 0)),
            pl.BlockSpec((4, 128, 512), lambda i: (0, 0, 0)),
            pl.BlockSpec((blk, C), lambda i: (i, 0)),
            pl.BlockSpec((C, C), lambda i: (0, 0)),
        ],
        out_specs=pl.BlockSpec((blk, 512), lambda i: (i, 0)),
        out_shape=jax.ShapeDtypeStruct((n_nodes, 512), jnp.float32),
    )(agg4, bmat, x, wsc)


def kernel(x, positions, senders, receivers, W0, W1, W2, W3, Wsc):
    n_nodes = x.shape[0]
    n_edges = senders.shape[0]
    quantum = NW * 128
    epad = ((n_edges + quantum - 1) // quantum) * quantum
    snd = jnp.pad(senders.astype(jnp.int32), (0, epad - n_edges))
    rcv = jnp.pad(receivers.astype(jnp.int32), (0, epad - n_edges))
    posx = positions[:, 0]
    posy = positions[:, 1]
    posz = positions[:, 2]
    fs, ylm = _edge_prep(x, posx, posy, posz, snd, rcv, n_edges)
    n_pad = ((n_nodes + 8 * NS - 1) // (8 * NS)) * (8 * NS)
    agg4 = _accumulate(rcv, fs, ylm, n_pad)
    bmat = _build_b(W0, W1, W2, W3)
    return _finish(agg4, x, bmat, Wsc)


# speedup vs baseline: 1.8560x; 1.0199x over previous
import functools

import jax
import jax.numpy as jnp
from jax import lax
from jax.experimental import pallas as pl
from jax.experimental.pallas import tpu as pltpu
import jax.experimental.pallas.tpu_sc as plsc

NC = 2
NS = 16
NW = NC * NS
L = 16
C = 32
DENOM = 16.0


def _rsqrt(v):
    i = plsc.bitcast(v, jnp.int32)
    i = jnp.int32(0x5F3759DF) - lax.shift_right_logical(i, 1)
    y = plsc.bitcast(i, jnp.float32)
    for _ in range(3):
        y = y * (1.5 - 0.5 * v * y * y)
    return y


def _ylm16(rx, ry, rz):
    n2 = rx * rx + ry * ry + rz * rz + 1e-9
    rinv = _rsqrt(n2)
    x = rx * rinv
    y = ry * rinv
    z = rz * rinv
    s3 = 3.0 ** 0.5
    s15 = 15.0 ** 0.5
    s5 = 5.0 ** 0.5
    c3m3 = 0.25 * (70.0 ** 0.5)
    c3m2 = 105.0 ** 0.5
    c3m1 = 0.25 * (42.0 ** 0.5)
    c30 = 0.5 * (7.0 ** 0.5)
    c32 = 0.5 * (105.0 ** 0.5)
    xx = x * x
    yy = y * y
    zz = z * z
    one = x * 0.0 + 1.0
    return [
        one,
        s3 * x, s3 * y, s3 * z,
        s15 * x * y,
        s15 * y * z,
        0.5 * s5 * (3.0 * zz - 1.0),
        s15 * x * z,
        0.5 * s15 * (xx - yy),
        c3m3 * y * (3.0 * xx - yy),
        c3m2 * x * y * z,
        c3m1 * y * (5.0 * zz - 1.0),
        c30 * (5.0 * zz * z - 3.0 * z),
        c3m1 * x * (5.0 * zz - 1.0),
        c32 * z * (xx - yy),
        c3m3 * x * (xx - 3.0 * yy),
    ]


def _edge_prep(x, posx, posy, posz, snd, rcv, n_edges):
    n_nodes = x.shape[0]
    epad = snd.shape[0]
    ew = epad // NW
    nblk = ew // 128
    mesh = plsc.VectorSubcoreMesh(
        core_axis_name="c", subcore_axis_name="s",
        num_cores=NC, num_subcores=NS)

    @functools.partial(
        pl.kernel,
        out_type=(jax.ShapeDtypeStruct((epad * C,), jnp.float32),
                  jax.ShapeDtypeStruct((epad * L,), jnp.float32)),
        mesh=mesh,
        compiler_params=pltpu.CompilerParams(
            needs_layout_passes=False, use_tc_tiling_on_sc=False),
        scratch_types=[
            pltpu.VMEM((n_nodes,), jnp.float32),
            pltpu.VMEM((n_nodes,), jnp.float32),
            pltpu.VMEM((n_nodes,), jnp.float32),
            pltpu.VMEM((128,), jnp.int32),
            pltpu.VMEM((128,), jnp.int32),
            pltpu.VMEM((128,), jnp.int32),
            pltpu.VMEM((128,), jnp.int32),
            pltpu.VMEM((128, C), jnp.float32),
            pltpu.VMEM((128, C), jnp.float32),
            pltpu.VMEM((128 * C,), jnp.float32),
            pltpu.VMEM((128 * C,), jnp.float32),
            pltpu.VMEM((128 * L,), jnp.float32),
            pltpu.VMEM((128 * L,), jnp.float32),
            pltpu.SemaphoreType.DMA,
            pltpu.SemaphoreType.DMA,
            pltpu.SemaphoreType.DMA,
            pltpu.SemaphoreType.DMA,
            pltpu.SemaphoreType.DMA,
            pltpu.SemaphoreType.DMA,
        ],
    )
    def prep(x_hbm, px_hbm, py_hbm, pz_hbm, snd_hbm, rcv_hbm,
             fs_hbm, ylm_hbm,
             px_v, py_v, pz_v, snd_a, rcv_a, snd_b, rcv_b,
             fs_a, fs_b, fsf_a, fsf_b, ylm_a, ylm_b,
             in_a, in_b, g_a, g_b, o_a, o_b):
        cid = lax.axis_index("c")
        sid = lax.axis_index("s")
        wid = sid * NC + cid
        base = wid * ew
        pltpu.sync_copy(px_hbm, px_v)
        pltpu.sync_copy(py_hbm, py_v)
        pltpu.sync_copy(pz_hbm, pz_v)
        lanes = lax.iota(jnp.int32, L)

        def start_in(j, snd_v, rcv_v, sem):
            off = base + j * 128
            return (
                pltpu.async_copy(snd_hbm.at[pl.ds(off, 128)],
                                 snd_v, sem),
                pltpu.async_copy(rcv_hbm.at[pl.ds(off, 128)],
                                 rcv_v, sem),
            )

        def compute(off, snd_v, rcv_v, ylmb):
            for k in range(8):
                sidx = snd_v[pl.ds(k * L, L)]
                ridx = rcv_v[pl.ds(k * L, L)]
                rx = plsc.load_gather(px_v, [ridx]) - plsc.load_gather(px_v, [sidx])
                ry = plsc.load_gather(py_v, [ridx]) - plsc.load_gather(py_v, [sidx])
                rz = plsc.load_gather(pz_v, [ridx]) - plsc.load_gather(pz_v, [sidx])
                comps = _ylm16(rx, ry, rz)
                gidx = off + k * L + lanes
                valid = jnp.where(gidx < n_edges, 1.0, 0.0)
                rowbase = (k * L + lanes) * L
                for ci, comp in enumerate(comps):
                    plsc.store_scatter(ylmb, [rowbase + ci], comp * valid)

        def repack(fsb, fsf):
            def row(e, carry):
                fsf[pl.ds(e * C, L)] = fsb[e, pl.ds(0, L)]
                fsf[pl.ds(e * C + L, L)] = fsb[e, pl.ds(L, L)]
                return carry

            lax.fori_loop(0, 128, row, 0)

        def start_out(j, fsf, ylmb, sem):
            off = base + j * 128
            return (
                pltpu.async_copy(fsf, fs_hbm.at[pl.ds(off * C, 128 * C)],
                                 sem),
                pltpu.async_copy(ylmb,
                                 ylm_hbm.at[pl.ds(off * L, 128 * L)],
                                 sem),
            )

        def body(jj, carry):
            j0 = jj * 2
            d0 = start_in(j0, snd_a, rcv_a, in_a)
            d1 = start_in(j0 + 1, snd_b, rcv_b, in_b)
            for d in d0:
                d.wait()
            ga = pltpu.async_copy(x_hbm.at[snd_a], fs_a, g_a)
            compute(base + j0 * 128, snd_a, rcv_a, ylm_a)
            for d in d1:
                d.wait()
            gb = pltpu.async_copy(x_hbm.at[snd_b], fs_b, g_b)
            ga.wait()
            repack(fs_a, fsf_a)
            oa = start_out(j0, fsf_a, ylm_a, o_a)
            compute(base + (j0 + 1) * 128, snd_b, rcv_b, ylm_b)
            gb.wait()
            repack(fs_b, fsf_b)
            ob = start_out(j0 + 1, fsf_b, ylm_b, o_b)
            for d in oa:
                d.wait()
            for d in ob:
                d.wait()
            return carry

        lax.fori_loop(0, nblk // 2, body, 0)

    return prep(x, posx, posy, posz, snd, rcv)


def _accumulate(rcv, fs, ylm, n_pad):
    epad = rcv.shape[0]
    et = epad // NS
    be = 80
    nblk = et // be
    rpt = n_pad // NS
    mesh = plsc.VectorSubcoreMesh(
        core_axis_name="c", subcore_axis_name="s",
        num_cores=NC, num_subcores=NS)

    @functools.partial(
        pl.kernel,
        out_type=jax.ShapeDtypeStruct((4, n_pad, 128), jnp.float32),
        mesh=mesh,
        compiler_params=pltpu.CompilerParams(needs_layout_passes=False),
        scratch_types=[
            pltpu.VMEM((be,), jnp.int32),
            pltpu.VMEM((be,), jnp.int32),
            pltpu.VMEM((be,), jnp.int32),
            pltpu.VMEM((be,), jnp.int32),
            pltpu.VMEM((be * C,), jnp.float32),
            pltpu.VMEM((be * C,), jnp.float32),
            pltpu.VMEM((be * L,), jnp.float32),
            pltpu.VMEM((be * L,), jnp.float32),
            pltpu.VMEM((be, 128), jnp.float32),
            pltpu.VMEM((be, 128), jnp.float32),
            pltpu.VMEM_SHARED((n_pad, 128), jnp.float32),
            pltpu.SemaphoreType.DMA,
            pltpu.SemaphoreType.DMA,
            pltpu.SemaphoreType.DMA,
            pltpu.SemaphoreType.DMA,
        ],
    )
    def acc(rcv_hbm, fs_hbm, ylm_hbm, out_hbm,
            rcv0, rcv1, rcv2, rcv3, fs_a, fs_b, ylm_a, ylm_b,
            msg_a, msg_b, agg_sh, in_a, in_b, ss_a, ss_b):
        cid = lax.axis_index("c")
        sid = lax.axis_index("s")
        tbase = sid * et
        rows0 = sid * rpt
        lanes = lax.iota(jnp.int32, L)
        zeros = jnp.zeros((L,), jnp.float32)

        def clear_my_rows():
            def zrow(r, carry):
                for cc in range(8):
                    plsc.store_scatter(
                        msg_a, [jnp.full((L,), r, jnp.int32),
                                cc * L + lanes],
                        zeros)
                return carry

            lax.fori_loop(0, be, zrow, 0)
            q, rem = divmod(rpt, be)
            for i in range(q):
                pltpu.sync_copy(msg_a,
                                agg_sh.at[pl.ds(rows0 + i * be, be)])
            if rem:
                pltpu.sync_copy(msg_a.at[pl.ds(0, rem)],
                                agg_sh.at[pl.ds(rows0 + q * be, rem)])

        clear_my_rows()
        plsc.subcore_barrier()

        for p in range(2):
            g = cid * 2 + p

            def start_in(j, rcv_v, fs_v, ylm_v, sem):
                off = tbase + j * be
                return (
                    pltpu.async_copy(rcv_hbm.at[pl.ds(off, be)],
                                     rcv_v, sem),
                    pltpu.async_copy(fs_hbm.at[pl.ds(off * C, be * C)],
                                     fs_v, sem),
                    pltpu.async_copy(ylm_hbm.at[pl.ds(off * L, be * L)],
                                     ylm_v, sem),
                )

            def wait_in(descs):
                for d in descs:
                    d.wait()

            def compute(fs_v, ylm_v, msg_v):
                @plsc.parallel_loop(0, be, unroll=4)
                def edge(e):
                    yrow = ylm_v[pl.ds(e * L, L)]
                    fsrow = fs_v[pl.ds(e * C + cid * L, L)]
                    for c8 in range(8):
                        s = fsrow[p * 8 + c8]
                        msg_v[e, pl.ds(c8 * L, L)] = s * yrow

            def scat(msg_v, rcv_v, sem):
                return pltpu.async_copy(msg_v, agg_sh.at[rcv_v],
                                        sem, add=True)

            def blk(jj, carry):
                j0 = jj * 4
                d0 = start_in(j0, rcv0, fs_a, ylm_a, in_a)
                d1 = start_in(j0 + 1, rcv1, fs_b, ylm_b, in_b)
                wait_in(d0)
                compute(fs_a, ylm_a, msg_a)
                s0 = scat(msg_a, rcv0, ss_a)
                d2 = start_in(j0 + 2, rcv2, fs_a, ylm_a, in_a)
                wait_in(d1)
                compute(fs_b, ylm_b, msg_b)
                s1 = scat(msg_b, rcv1, ss_b)
                d3 = start_in(j0 + 3, rcv3, fs_b, ylm_b, in_b)
                s0.wait()
                wait_in(d2)
                compute(fs_a, ylm_a, msg_a)
                s2 = scat(msg_a, rcv2, ss_a)
                s1.wait()
                wait_in(d3)
                compute(fs_b, ylm_b, msg_b)
                s3 = scat(msg_b, rcv3, ss_b)
                s2.wait()
                s3.wait()
                return carry

            lax.fori_loop(0, nblk // 4, blk, 0)
            plsc.subcore_barrier()
            q, rem = divmod(rpt, 128)
            for i in range(q):
                pltpu.sync_copy(agg_sh.at[pl.ds(rows0 + i * 128, 128)],
                                out_hbm.at[g, pl.ds(rows0 + i * 128, 128)])
            if rem:
                pltpu.sync_copy(agg_sh.at[pl.ds(rows0 + q * 128, rem)],
                                out_hbm.at[g, pl.ds(rows0 + q * 128, rem)])
            if p == 0:
                clear_my_rows()
            plsc.subcore_barrier()

    return acc(rcv, fs, ylm)


def _build_b(w0, w1, w2, w3):
    import numpy as np
    s = (1.0 / DENOM) * (1.0 / (C ** 0.5))
    parts = []
    for moff, d, w in ((0, 1, w0), (1, 3, w1), (4, 5, w2), (9, 7, w3)):
        e = np.zeros((L, d), np.float32)
        for i in range(d):
            e[moff + i, i] = 1.0
        wg = (s * w).reshape(4, 8, C)
        parts.append(
            jnp.einsum('gco,ri->gcroi', wg, jnp.asarray(e))
            .reshape(4, 128, C * d))
    return jnp.concatenate(parts, axis=-1)


def _finish(agg4, x, bmat, wsc):
    n_nodes = x.shape[0]
    blk = 400
    grid = (n_nodes // blk,)
    inv = 1.0 / (C ** 0.5)

    def body(agg_ref, b_ref, x_ref, wsc_ref, out_ref):
        acc = jnp.zeros((blk, 512), jnp.float32)
        for p in range(4):
            acc = acc + jnp.dot(agg_ref[p], b_ref[p],
                                preferred_element_type=jnp.float32)
        sc0 = jnp.dot(x_ref[...], wsc_ref[...],
                      preferred_element_type=jnp.float32) * inv
        out_ref[...] = acc
        out_ref[:, 0:C] = acc[:, 0:C] + sc0

    return pl.pallas_call(
        body,
        grid=grid,
        in_specs=[
            pl.BlockSpec((4, blk, 128), lambda i: (0, i, 0)),
            pl.BlockSpec((4, 128, 512), lambda i: (0, 0, 0)),
            pl.BlockSpec((blk, C), lambda i: (i, 0)),
            pl.BlockSpec((C, C), lambda i: (0, 0)),
        ],
        out_specs=pl.BlockSpec((blk, 512), lambda i: (i, 0)),
        out_shape=jax.ShapeDtypeStruct((n_nodes, 512), jnp.float32),
    )(agg4, bmat, x, wsc)


def kernel(x, positions, senders, receivers, W0, W1, W2, W3, Wsc):
    n_nodes = x.shape[0]
    n_edges = senders.shape[0]
    quantum = NW * 128
    epad = ((n_edges + quantum - 1) // quantum) * quantum
    snd = jnp.pad(senders.astype(jnp.int32), (0, epad - n_edges))
    rcv = jnp.pad(receivers.astype(jnp.int32), (0, epad - n_edges))
    posx = positions[:, 0]
    posy = positions[:, 1]
    posz = positions[:, 2]
    fs, ylm = _edge_prep(x, posx, posy, posz, snd, rcv, n_edges)
    n_pad = ((n_nodes + 8 * NS - 1) // (8 * NS)) * (8 * NS)
    agg4 = _accumulate(rcv, fs, ylm, n_pad)
    bmat = _build_b(W0, W1, W2, W3)
    return _finish(agg4, x, bmat, Wsc)
